# Initial kernel scaffold; baseline (speedup 1.0000x reference)
import jax
import jax.numpy as jnp
from jax.experimental import pallas as pl


def kernel(x, Wr1, g1, b1, Wr2, g2, b2, Wp, gp, bp, Wc, gc, bc):
    raise NotImplementedError("write your pallas kernel here")


# speedup vs baseline: 6.6466x; 6.6466x over previous
import jax
import jax.numpy as jnp
from jax.experimental import pallas as pl
from jax.experimental.pallas import tpu as pltpu

E = 8
K = 2
CIN = 96
INIT = 48
RED = 12
EPS = 1e-5
H = 224
W = 224

R = 28
T = H // R
P = 6
NC = (R + 2) // P
KP = 9 * 128

RT = 32
TR = H // RT


def _routing_kernel(x_ref, w1_ref, s1_ref, b1_ref, w2_ref, s2_ref, b2_ref,
                    idx_ref, val_ref, acc_ref):
    t = pl.program_id(0)
    part = jnp.sum(x_ref[...], axis=(2, 3))

    @pl.when(t == 0)
    def _():
        acc_ref[...] = part

    @pl.when(t != 0)
    def _():
        acc_ref[...] = acc_ref[...] + part

    @pl.when(t == TR - 1)
    def _():
        pooled = acc_ref[...] * (1.0 / (H * W))
        h = jnp.dot(pooled, w1_ref[...], preferred_element_type=jnp.float32)
        h = h * s1_ref[...] + b1_ref[...]
        h = h * jax.nn.sigmoid(h)
        lg = jnp.dot(h, w2_ref[...], preferred_element_type=jnp.float32)
        lg = lg * s2_ref[...] + b2_ref[...]
        m = jnp.max(lg, axis=1, keepdims=True)
        ex = jnp.exp(lg - m)
        p = ex / jnp.sum(ex, axis=1, keepdims=True)
        iota = jax.lax.broadcasted_iota(jnp.int32, (2, E), 1)
        v0 = jnp.max(p, axis=1, keepdims=True)
        i0 = jnp.min(jnp.where(p == v0, iota, E), axis=1, keepdims=True)
        p2 = jnp.where(iota == i0, -1.0, p)
        v1 = jnp.max(p2, axis=1, keepdims=True)
        i1 = jnp.min(jnp.where(p2 == v1, iota, E), axis=1, keepdims=True)
        ssum = v0 + v1 + 1e-6
        idx_ref[...] = jnp.concatenate([i0, i1], axis=1)
        val_ref[...] = jnp.concatenate([v0 / ssum, v1 / ssum], axis=1)


def _conv_kernel(idx_ref, val_ref, xa_ref, xb_ref, w8_ref, bp_ref, wd_ref,
                 bc_ref, out_ref, xs_ref, x1_ref, pt_ref):
    b = pl.program_id(0)
    t = pl.program_id(1)
    e0 = idx_ref[b, 0]
    e1 = idx_ref[b, 1]
    v0 = val_ref[b, 0]
    v1 = val_ref[b, 1]

    xs_ref[0:R] = xa_ref[0]
    xs_ref[R:R + 4] = xb_ref[0]

    wsel = jnp.concatenate([w8_ref[e0], w8_ref[e1]], axis=1)
    sh1 = jnp.concatenate([bp_ref[e0], bp_ref[e1]], axis=1)

    zpad = jnp.zeros((P * W, 128 - CIN), jnp.float32)
    for tap in range(9):
        pt_ref[:, tap * 128 + CIN:(tap + 1) * 128] = zpad

    for c in range(NC):
        for p_ in range(P):
            s1 = c * P + p_
            for dy in range(3):
                for dx in range(3):
                    tap = dy * 3 + dx
                    pt_ref[p_ * W:(p_ + 1) * W, tap * 128:tap * 128 + CIN] = (
                        xs_ref[s1 + dy, dx:dx + W, :])
        raw = jnp.dot(pt_ref[...], wsel, preferred_element_type=jnp.float32)
        y = raw + sh1
        y = y * jax.nn.sigmoid(y)
        y3 = y.reshape(P, W, 2 * INIT)
        jrow = (t * R - 1 + c * P
                + jax.lax.broadcasted_iota(jnp.int32, (P, 1, 1), 0))
        y3 = jnp.where((jrow >= 0) & (jrow < H), y3, 0.0)
        x1_ref[c * P:(c + 1) * P, 1:1 + W, :] = y3
    zc = jnp.zeros((R + 2, 2 * INIT), jnp.float32)
    x1_ref[:, 0, :] = zc
    x1_ref[:, 1 + W, :] = zc

    wd = jnp.concatenate([wd_ref[e0], wd_ref[e1]], axis=1)
    sh2 = jnp.concatenate([bc_ref[e0], bc_ref[e1]], axis=1)
    acc = None
    for dy in range(3):
        for dx in range(3):
            tap = dy * 3 + dx
            term = x1_ref[dy:dy + R, dx:dx + W, :] * wd[tap][None, None, :]
            acc = term if acc is None else acc + term
    y2 = acc + sh2[0][None, None, :]
    y2 = y2 * jax.nn.sigmoid(y2)

    x1c = x1_ref[1:R + 1, 1:1 + W, :]
    top = v0 * x1c[:, :, 0:INIT] + v1 * x1c[:, :, INIT:2 * INIT]
    bot = v0 * y2[:, :, 0:INIT] + v1 * y2[:, :, INIT:2 * INIT]
    out_ref[0] = jnp.concatenate([top, bot], axis=2)


def kernel(x, Wr1, g1, b1, Wr2, g2, b2, Wp, gp, bp, Wc, gc, bc):
    B = x.shape[0]
    inv = 1.0 / jnp.sqrt(1.0 + EPS)

    idx, vals = pl.pallas_call(
        _routing_kernel,
        grid=(TR,),
        in_specs=[
            pl.BlockSpec((B, CIN, RT, W), lambda t: (0, 0, t, 0)),
            pl.BlockSpec((CIN, RED), lambda t: (0, 0)),
            pl.BlockSpec((1, RED), lambda t: (0, 0)),
            pl.BlockSpec((1, RED), lambda t: (0, 0)),
            pl.BlockSpec((RED, E), lambda t: (0, 0)),
            pl.BlockSpec((1, E), lambda t: (0, 0)),
            pl.BlockSpec((1, E), lambda t: (0, 0)),
        ],
        out_specs=[
            pl.BlockSpec((B, K), lambda t: (0, 0)),
            pl.BlockSpec((B, K), lambda t: (0, 0)),
        ],
        out_shape=[
            jax.ShapeDtypeStruct((B, K), jnp.int32),
            jax.ShapeDtypeStruct((B, K), jnp.float32),
        ],
        scratch_shapes=[pltpu.VMEM((B, CIN), jnp.float32)],
    )(x, Wr1.T, (g1 * inv)[None, :], b1[None, :],
      Wr2.T, (g2 * inv)[None, :], b2[None, :])

    sp = gp * inv
    w8 = (Wp * sp[:, :, None, None, None]).transpose(0, 3, 4, 2, 1)
    w8 = jnp.pad(w8, ((0, 0), (0, 0), (0, 0), (0, 128 - CIN), (0, 0)))
    w8 = w8.reshape(E, KP, INIT)
    sc = gc * inv
    wd8 = (Wc[:, :, 0] * sc[:, :, None, None]).transpose(0, 2, 3, 1)
    wd8 = wd8.reshape(E, 9, INIT)
    bp3 = bp[:, None, :]
    bc3 = bc[:, None, :]

    xp = jnp.pad(x.transpose(0, 2, 3, 1), ((0, 0), (2, 6), (1, 1), (0, 0)))

    out_nhwc = pl.pallas_call(
        _conv_kernel,
        grid=(B, T),
        in_specs=[
            pl.BlockSpec(memory_space=pltpu.SMEM),
            pl.BlockSpec(memory_space=pltpu.SMEM),
            pl.BlockSpec((1, R, W + 2, CIN), lambda b, t: (b, t, 0, 0)),
            pl.BlockSpec((1, 4, W + 2, CIN), lambda b, t: (b, 7 * (t + 1), 0, 0)),
            pl.BlockSpec((E, KP, INIT), lambda b, t: (0, 0, 0)),
            pl.BlockSpec((E, 1, INIT), lambda b, t: (0, 0, 0)),
            pl.BlockSpec((E, 9, INIT), lambda b, t: (0, 0, 0)),
            pl.BlockSpec((E, 1, INIT), lambda b, t: (0, 0, 0)),
        ],
        out_specs=pl.BlockSpec((1, R, W, 2 * INIT), lambda b, t: (b, t, 0, 0)),
        out_shape=jax.ShapeDtypeStruct((B, H, W, 2 * INIT), jnp.float32),
        scratch_shapes=[
            pltpu.VMEM((R + 4, W + 2, CIN), jnp.float32),
            pltpu.VMEM((R + 2, W + 2, 2 * INIT), jnp.float32),
            pltpu.VMEM((P * W, KP), jnp.float32),
        ],
    )(idx, vals, xp, xp, w8, bp3, wd8, bc3)

    return out_nhwc.transpose(0, 3, 1, 2)


# speedup vs baseline: 13.6994x; 2.0611x over previous
import jax
import jax.numpy as jnp
from jax.experimental import pallas as pl
from jax.experimental.pallas import tpu as pltpu

E = 8
K = 2
CIN = 96
INIT = 48
RED = 12
EPS = 1e-5
H = 224
W = 224

R = 32
T = H // R
ST = 256
NX1 = R + 2
L1 = NX1 * ST
L2 = R * ST

RT = 32
TR = H // RT


def _routing_kernel(x_ref, w1_ref, s1_ref, b1_ref, w2_ref, s2_ref, b2_ref,
                    idx_ref, val_ref, acc_ref):
    t = pl.program_id(0)
    part = jnp.sum(x_ref[...], axis=(2, 3))

    @pl.when(t == 0)
    def _():
        acc_ref[...] = part

    @pl.when(t != 0)
    def _():
        acc_ref[...] = acc_ref[...] + part

    @pl.when(t == TR - 1)
    def _():
        pooled = acc_ref[...] * (1.0 / (H * W))
        h = jnp.dot(pooled, w1_ref[...], preferred_element_type=jnp.float32)
        h = h * s1_ref[...] + b1_ref[...]
        h = h * jax.nn.sigmoid(h)
        lg = jnp.dot(h, w2_ref[...], preferred_element_type=jnp.float32)
        lg = lg * s2_ref[...] + b2_ref[...]
        m = jnp.max(lg, axis=1, keepdims=True)
        ex = jnp.exp(lg - m)
        p = ex / jnp.sum(ex, axis=1, keepdims=True)
        iota = jax.lax.broadcasted_iota(jnp.int32, (2, E), 1)
        v0 = jnp.max(p, axis=1, keepdims=True)
        i0 = jnp.min(jnp.where(p == v0, iota, E), axis=1, keepdims=True)
        p2 = jnp.where(iota == i0, -1.0, p)
        v1 = jnp.max(p2, axis=1, keepdims=True)
        i1 = jnp.min(jnp.where(p2 == v1, iota, E), axis=1, keepdims=True)
        ssum = v0 + v1 + 1e-6
        idx_ref[...] = jnp.concatenate([i0, i1], axis=1)
        val_ref[...] = jnp.concatenate([v0 / ssum, v1 / ssum], axis=1)


def _shift_p1(a):
    return jnp.concatenate([jnp.zeros((a.shape[0], 1), a.dtype), a[:, :-1]],
                           axis=1)


def _shift_m1(a):
    return jnp.concatenate([a[:, 1:], jnp.zeros((a.shape[0], 1), a.dtype)],
                           axis=1)


def _conv_kernel(idx_ref, val_ref, xa_ref, xb_ref, xc_ref, w9_ref, bp_ref,
                 wd_ref, bc_ref, out_ref, xs_ref, x1_ref):
    b = pl.program_id(0)
    t = pl.program_id(1)
    e0 = idx_ref[b, 0]
    e1 = idx_ref[b, 1]
    v0 = val_ref[b, 0]
    v1 = val_ref[b, 1]

    @pl.when((b == 0) & (t == 0))
    def _():
        q = jnp.bitwise_and(
            jax.lax.broadcasted_iota(jnp.int32, (CIN, xs_ref.shape[1]), 1),
            ST - 1)
        border = (q == 0) | (q == 1 + W)
        xs_ref[...] = jnp.where(border, 0.0, xs_ref[...])

    for i in range(R):
        s = 3 + i
        xs_ref[:, s * ST + 1:s * ST + 1 + W] = xa_ref[0, :, i, :]

    @pl.when(t > 0)
    def _():
        xs_ref[:, 1 * ST + 1:1 * ST + 1 + W] = xb_ref[0, :, 6, :]
        xs_ref[:, 2 * ST + 1:2 * ST + 1 + W] = xb_ref[0, :, 7, :]

    @pl.when(t == 0)
    def _():
        z = jnp.zeros((CIN, W), jnp.float32)
        xs_ref[:, 1 * ST + 1:1 * ST + 1 + W] = z
        xs_ref[:, 2 * ST + 1:2 * ST + 1 + W] = z

    @pl.when(t < T - 1)
    def _():
        xs_ref[:, 35 * ST + 1:35 * ST + 1 + W] = xc_ref[0, :, 0, :]
        xs_ref[:, 36 * ST + 1:36 * ST + 1 + W] = xc_ref[0, :, 1, :]

    @pl.when(t == T - 1)
    def _():
        z = jnp.zeros((CIN, W), jnp.float32)
        xs_ref[:, 35 * ST + 1:35 * ST + 1 + W] = z
        xs_ref[:, 36 * ST + 1:36 * ST + 1 + W] = z

    w9 = jnp.concatenate([w9_ref[e0], w9_ref[e1]], axis=1)
    sh1 = jnp.concatenate([bp_ref[e0], bp_ref[e1]], axis=0)
    wd = jnp.concatenate([wd_ref[e0], wd_ref[e1]], axis=0)
    sh2 = jnp.concatenate([bc_ref[e0], bc_ref[e1]], axis=0)

    pdx = []
    for dx in range(3):
        acc = None
        for dy in range(3):
            s_dy = xs_ref[:, (1 + dy) * ST:(1 + dy) * ST + L1]
            term = jnp.dot(w9[dy * 3 + dx], s_dy,
                           preferred_element_type=jnp.float32)
            acc = term if acc is None else acc + term
        pdx.append(acc)
    raw = _shift_p1(pdx[0]) + pdx[1] + _shift_m1(pdx[2]) + sh1
    y1 = raw * jax.nn.sigmoid(raw)

    q = jax.lax.broadcasted_iota(jnp.int32, (2 * INIT, L1), 1)
    qc = jnp.bitwise_and(q, ST - 1)
    jrow = t * R - 1 + jnp.right_shift(q, 8)
    keep = (qc >= 1) & (qc < 1 + W) & (jrow >= 0) & (jrow < H)
    x1_ref[:, ST:ST + L1] = jnp.where(keep, y1, 0.0)

    qdx = []
    for dx in range(3):
        acc = None
        for dy in range(3):
            s_dy = x1_ref[:, (1 + dy) * ST:(1 + dy) * ST + L2]
            term = wd[:, dy * 3 + dx][:, None] * s_dy
            acc = term if acc is None else acc + term
        qdx.append(acc)
    y2r = _shift_p1(qdx[0]) + qdx[1] + _shift_m1(qdx[2]) + sh2
    y2 = y2r * jax.nn.sigmoid(y2r)

    for r in range(R):
        a1 = x1_ref[:, (r + 2) * ST + 1:(r + 2) * ST + 1 + W]
        b2_ = y2[:, r * ST + 1:r * ST + 1 + W]
        top = v0 * a1[0:INIT] + v1 * a1[INIT:2 * INIT]
        bot = v0 * b2_[0:INIT] + v1 * b2_[INIT:2 * INIT]
        out_ref[0, :, r, :] = jnp.concatenate([top, bot], axis=0)


def kernel(x, Wr1, g1, b1, Wr2, g2, b2, Wp, gp, bp, Wc, gc, bc):
    B = x.shape[0]
    inv = 1.0 / jnp.sqrt(1.0 + EPS)

    idx, vals = pl.pallas_call(
        _routing_kernel,
        grid=(TR,),
        in_specs=[
            pl.BlockSpec((B, CIN, RT, W), lambda t: (0, 0, t, 0)),
            pl.BlockSpec((CIN, RED), lambda t: (0, 0)),
            pl.BlockSpec((1, RED), lambda t: (0, 0)),
            pl.BlockSpec((1, RED), lambda t: (0, 0)),
            pl.BlockSpec((RED, E), lambda t: (0, 0)),
            pl.BlockSpec((1, E), lambda t: (0, 0)),
            pl.BlockSpec((1, E), lambda t: (0, 0)),
        ],
        out_specs=[
            pl.BlockSpec((B, K), lambda t: (0, 0)),
            pl.BlockSpec((B, K), lambda t: (0, 0)),
        ],
        out_shape=[
            jax.ShapeDtypeStruct((B, K), jnp.int32),
            jax.ShapeDtypeStruct((B, K), jnp.float32),
        ],
        scratch_shapes=[pltpu.VMEM((B, CIN), jnp.float32)],
    )(x, Wr1.T, (g1 * inv)[None, :], b1[None, :],
      Wr2.T, (g2 * inv)[None, :], b2[None, :])

    sp = gp * inv
    w9 = (Wp * sp[:, :, None, None, None]).transpose(0, 3, 4, 1, 2)
    w9 = w9.reshape(E, 9, INIT, CIN)
    sc = gc * inv
    wd9 = (Wc[:, :, 0] * sc[:, :, None, None]).reshape(E, INIT, 9)
    bp3 = bp[:, :, None]
    bc3 = bc[:, :, None]

    out = pl.pallas_call(
        _conv_kernel,
        grid=(B, T),
        in_specs=[
            pl.BlockSpec(memory_space=pltpu.SMEM),
            pl.BlockSpec(memory_space=pltpu.SMEM),
            pl.BlockSpec((1, CIN, R, W), lambda b, t: (b, 0, t, 0)),
            pl.BlockSpec((1, CIN, 8, W),
                         lambda b, t: (b, 0, jnp.maximum(4 * t - 1, 0), 0)),
            pl.BlockSpec((1, CIN, 8, W),
                         lambda b, t: (b, 0, jnp.minimum(4 * t + 4, 27), 0)),
            pl.BlockSpec((E, 9, INIT, CIN), lambda b, t: (0, 0, 0, 0)),
            pl.BlockSpec((E, INIT, 1), lambda b, t: (0, 0, 0)),
            pl.BlockSpec((E, INIT, 9), lambda b, t: (0, 0, 0)),
            pl.BlockSpec((E, INIT, 1), lambda b, t: (0, 0, 0)),
        ],
        out_specs=pl.BlockSpec((1, 2 * INIT, R, W), lambda b, t: (b, 0, t, 0)),
        out_shape=jax.ShapeDtypeStruct((B, 2 * INIT, H, W), jnp.float32),
        scratch_shapes=[
            pltpu.VMEM((CIN, 38 * ST), jnp.float32),
            pltpu.VMEM((2 * INIT, 35 * ST), jnp.float32),
        ],
    )(idx, vals, x, x, x, w9, bp3, wd9, bc3)

    return out
